# baseline (speedup 1.0000x reference)
import functools

import jax
import jax.numpy as jnp
from jax import lax
from jax.experimental import pallas as pl
from jax.experimental.pallas import tpu as pltpu
from jax.experimental.pallas import tpu_sc as plsc

D_MODEL = 64
BATCH = 4096
SEQ = 200

NUM_CORES = 2
NUM_SUBCORES = 16
NUM_WORKERS = NUM_CORES * NUM_SUBCORES
B_PER_W = BATCH // NUM_WORKERS

_MESH = plsc.VectorSubcoreMesh(core_axis_name="c", subcore_axis_name="s")


@functools.partial(
    pl.kernel,
    mesh=_MESH,
    out_type=jax.ShapeDtypeStruct((BATCH, SEQ, D_MODEL), jnp.float32),
    scratch_types=[
        pltpu.VMEM((B_PER_W, SEQ), jnp.int32),
        pltpu.VMEM((SEQ, D_MODEL), jnp.float32),
        pltpu.VMEM((SEQ, D_MODEL), jnp.float32),
        pltpu.SemaphoreType.DMA,
        pltpu.SemaphoreType.DMA,
        pltpu.SemaphoreType.DMA,
        pltpu.SemaphoreType.DMA,
    ],
    compiler_params=pltpu.CompilerParams(use_tc_tiling_on_sc=False),
)
def _gather_rows(idx_hbm, table_hbm, out_hbm, idx_v, rb0, rb1, g0, g1, s0, s1):
    wid = lax.axis_index("s") * NUM_CORES + lax.axis_index("c")
    base = wid * B_PER_W
    pltpu.sync_copy(idx_hbm.at[pl.ds(base, B_PER_W)], idx_v)

    rb = (rb0, rb1)
    gsem = (g0, g1)
    ssem = (s0, s1)

    def gather(i, b):
        return pltpu.make_async_copy(table_hbm.at[idx_v.at[i]], rb[b], gsem[b])

    def store(i, b):
        return pltpu.make_async_copy(rb[b], out_hbm.at[base + i], ssem[b])

    gather(0, 0).start()
    for i in range(B_PER_W):
        b = i % 2
        gather(i, b).wait()
        if i >= 1:
            store(i - 1, 1 - b).wait()
        if i + 1 < B_PER_W:
            gather(i + 1, 1 - b).start()
        store(i, b).start()
    store(B_PER_W - 1, (B_PER_W - 1) % 2).wait()


def kernel(x, W):
    return _gather_rows(x.astype(jnp.int32), W)
